# Initial kernel scaffold; baseline (speedup 1.0000x reference)
import jax
import jax.numpy as jnp
from jax.experimental import pallas as pl


def kernel(src_ft, dst_ft, edge_p, edge_index, W_pos_w, W_pos_b):
    raise NotImplementedError("write your pallas kernel here")


# speedup vs baseline: 1.3308x; 1.3308x over previous
import jax
import jax.numpy as jnp
from jax.experimental import pallas as pl

N_NODES = 10000
NEG_SLOPE = 0.2


def _matmul_kernel(a_ref, w_ref, b_ref, o_ref):
    o_ref[...] = jnp.dot(a_ref[...], w_ref[...],
                         preferred_element_type=jnp.float32) + b_ref[...]


def _linear(a, wt, b):
    E = a.shape[0]
    BLK = 1000
    grid = (E // BLK,)
    return pl.pallas_call(
        _matmul_kernel,
        grid=grid,
        in_specs=[
            pl.BlockSpec((BLK, 16), lambda i: (i, 0)),
            pl.BlockSpec((16, 256), lambda i: (0, 0)),
            pl.BlockSpec((1, 256), lambda i: (0, 0)),
        ],
        out_specs=pl.BlockSpec((BLK, 256), lambda i: (i, 0)),
        out_shape=jax.ShapeDtypeStruct((E, 256), jnp.float32),
    )(a, wt, b)


def kernel(src_ft, dst_ft, edge_p, edge_index, W_pos_w, W_pos_b):
    src = edge_index[0]
    dst = edge_index[1]
    e = jnp.where(edge_p >= 0, edge_p, NEG_SLOPE * edge_p)
    ee = jnp.exp(e)
    den = jax.ops.segment_sum(ee, dst, num_segments=N_NODES)
    a = ee / den[dst]
    A = _linear(a, W_pos_w.T, W_pos_b[None, :])
    m = src_ft[src] * A
    out = jax.ops.segment_sum(m, dst, num_segments=N_NODES)
    return out[:, None, :]


# speedup vs baseline: 1.9340x; 1.4532x over previous
import functools

import jax
import jax.numpy as jnp
from jax import lax
from jax.experimental import pallas as pl
from jax.experimental.pallas import tpu as pltpu
from jax.experimental.pallas import tpu_sc as plsc

N = 10000
E = 160000
AUX = 16
OUT = 256
HALF = OUT // 2
NEG = 0.2

NC, NS, L = 2, 16, 16
NW = NC * NS
CH = 128
E_PAD = 163840
EPW = E_PAD // NW
EPT = E_PAD // NS
N_PAD = 10112
ROWS_PT = N_PAD // NS

_MESH = plsc.VectorSubcoreMesh(core_axis_name="c", subcore_axis_name="s")
_f32 = jnp.float32


def _leaky_exp(x):
    return jnp.exp(jnp.where(x >= 0, x, NEG * x))


@functools.partial(
    pl.kernel,
    out_type=jax.ShapeDtypeStruct((E_PAD, AUX), _f32),
    mesh=_MESH,
    scratch_types=[
        pltpu.VMEM_SHARED((N_PAD, AUX), _f32),
        pltpu.VMEM((CH, AUX), _f32),
        pltpu.VMEM((CH, AUX), _f32),
        pltpu.VMEM((CH,), jnp.int32),
        pltpu.SemaphoreType.DMA,
    ],
)
def _stage_a(p_hbm, dst_hbm, zero_hbm, a_hbm,
             den_sh, p_v, d_v, dst_v, sem):
    c = lax.axis_index("c")
    s = lax.axis_index("s")
    wid = s * NC + c
    row0 = s * ROWS_PT
    pltpu.sync_copy(zero_hbm.at[pl.ds(row0, ROWS_PT)],
                    den_sh.at[pl.ds(row0, ROWS_PT)])
    plsc.subcore_barrier()

    def chunk1(i, _):
        base = s * EPT + i * CH
        pltpu.sync_copy(p_hbm.at[pl.ds(base, CH)], p_v)
        pltpu.sync_copy(dst_hbm.at[pl.ds(base, CH)], dst_v)

        def row(j, _):
            p_v[j, :] = _leaky_exp(p_v[j, :])
            return 0

        lax.fori_loop(0, CH, row, 0)
        pltpu.sync_copy(p_v, den_sh.at[dst_v], add=True)
        return 0

    lax.fori_loop(0, EPT // CH, chunk1, 0)
    plsc.subcore_barrier()

    def chunk2(i, _):
        base = wid * EPW + i * CH
        pltpu.sync_copy(p_hbm.at[pl.ds(base, CH)], p_v)
        pltpu.sync_copy(dst_hbm.at[pl.ds(base, CH)], dst_v)
        pltpu.async_copy(den_sh.at[dst_v], d_v, sem).wait()

        def row(j, _):
            p_v[j, :] = _leaky_exp(p_v[j, :]) / d_v[j, :]
            return 0

        lax.fori_loop(0, CH, row, 0)
        pltpu.sync_copy(p_v, a_hbm.at[pl.ds(base, CH)])
        return 0

    lax.fori_loop(0, EPW // CH, chunk2, 0)


def _mm_body(a_ref, wt_ref, b_ref, lo_ref, hi_ref):
    r = jnp.dot(a_ref[...], wt_ref[...], preferred_element_type=_f32)
    r = r + b_ref[...]
    lo_ref[...] = r[:, :HALF]
    hi_ref[...] = r[:, HALF:]


def _linear(a, wt, b):
    BLK = 1024
    return pl.pallas_call(
        _mm_body,
        grid=(E_PAD // BLK,),
        in_specs=[
            pl.BlockSpec((BLK, AUX), lambda i: (i, 0)),
            pl.BlockSpec((AUX, OUT), lambda i: (0, 0)),
            pl.BlockSpec((1, OUT), lambda i: (0, 0)),
        ],
        out_specs=[
            pl.BlockSpec((BLK, HALF), lambda i: (i, 0)),
            pl.BlockSpec((BLK, HALF), lambda i: (i, 0)),
        ],
        out_shape=[
            jax.ShapeDtypeStruct((E_PAD, HALF), _f32),
            jax.ShapeDtypeStruct((E_PAD, HALF), _f32),
        ],
    )(a, wt, b)


@functools.partial(
    pl.kernel,
    out_type=(
        jax.ShapeDtypeStruct((N_PAD, HALF), _f32),
        jax.ShapeDtypeStruct((N_PAD, HALF), _f32),
    ),
    mesh=_MESH,
    scratch_types=[
        pltpu.VMEM_SHARED((N_PAD, HALF), _f32),
        pltpu.VMEM((CH, HALF), _f32),
        pltpu.VMEM((CH, HALF), _f32),
        pltpu.VMEM((CH,), jnp.int32),
        pltpu.VMEM((CH,), jnp.int32),
        pltpu.SemaphoreType.DMA,
    ],
)
def _stage_c(srclo_hbm, srchi_hbm, alo_hbm, ahi_hbm, src_hbm, dst_hbm,
             zero_hbm, outlo_hbm, outhi_hbm,
             acc_sh, rows_v, a_v, src_v, dst_v, sem):
    c = lax.axis_index("c")
    s = lax.axis_index("s")
    row0 = s * ROWS_PT
    pltpu.sync_copy(zero_hbm.at[pl.ds(row0, ROWS_PT)],
                    acc_sh.at[pl.ds(row0, ROWS_PT)])
    plsc.subcore_barrier()

    def run(tbl_hbm, a_half_hbm):
        def chunk(i, _):
            base = s * EPT + i * CH
            pltpu.sync_copy(src_hbm.at[pl.ds(base, CH)], src_v)
            pltpu.sync_copy(dst_hbm.at[pl.ds(base, CH)], dst_v)
            pltpu.async_copy(tbl_hbm.at[src_v], rows_v, sem).wait()
            pltpu.sync_copy(a_half_hbm.at[pl.ds(base, CH)], a_v)

            def row(j, _):
                for q in range(HALF // L):
                    sl = pl.ds(q * L, L)
                    rows_v[j, sl] = rows_v[j, sl] * a_v[j, sl]
                return 0

            lax.fori_loop(0, CH, row, 0)
            pltpu.sync_copy(rows_v, acc_sh.at[dst_v], add=True)
            return 0

        lax.fori_loop(0, EPT // CH, chunk, 0)

    @pl.when(c == 0)
    def _():
        run(srclo_hbm, alo_hbm)

    @pl.when(c == 1)
    def _():
        run(srchi_hbm, ahi_hbm)

    plsc.subcore_barrier()

    @pl.when(c == 0)
    def _():
        pltpu.sync_copy(acc_sh.at[pl.ds(row0, ROWS_PT)],
                        outlo_hbm.at[pl.ds(row0, ROWS_PT)])

    @pl.when(c == 1)
    def _():
        pltpu.sync_copy(acc_sh.at[pl.ds(row0, ROWS_PT)],
                        outhi_hbm.at[pl.ds(row0, ROWS_PT)])


def kernel(src_ft, dst_ft, edge_p, edge_index, W_pos_w, W_pos_b):
    src = edge_index[0]
    dst = edge_index[1]
    pad = E_PAD - E
    p_pad = jnp.pad(edge_p, ((0, pad), (0, 0)))
    src_pad = jnp.pad(src, (0, pad))
    dst_pad = jnp.pad(dst, (0, pad), constant_values=N)
    zero_aux = jnp.zeros((N_PAD, AUX), _f32)
    zero_half = jnp.zeros((N_PAD, HALF), _f32)

    a = _stage_a(p_pad, dst_pad, zero_aux)
    a_lo, a_hi = _linear(a, W_pos_w.T, W_pos_b[None, :])
    out_lo, out_hi = _stage_c(src_ft[:, :HALF], src_ft[:, HALF:],
                              a_lo, a_hi, src_pad, dst_pad, zero_half)
    out = jnp.concatenate([out_lo[:N], out_hi[:N]], axis=1)
    return out[:, None, :]


# speedup vs baseline: 2.8212x; 1.4588x over previous
import functools

import jax
import jax.numpy as jnp
from jax import lax
from jax.experimental import pallas as pl
from jax.experimental.pallas import tpu as pltpu
from jax.experimental.pallas import tpu_sc as plsc

N = 10000
E = 160000
AUX = 16
OUT = 256
HALF = OUT // 2
NEG = 0.2

NC, NS, L = 2, 16, 16
NW = NC * NS
CH = 128
CC = 64
BLK = 512
SUB = BLK // CH
E_PAD = 163840
EPW = E_PAD // NW
EPT = E_PAD // NS
NB1 = EPT // BLK
NB2 = EPW // BLK
NCC = EPT // CC
PACK = 16384
N_PAD = 10112
ROWS_PT = N_PAD // NS

_MESH = plsc.VectorSubcoreMesh(core_axis_name="c", subcore_axis_name="s")
_f32 = jnp.float32
_NOTC = pltpu.CompilerParams(use_tc_tiling_on_sc=False)


def _leaky_exp(x):
    return jnp.exp(jnp.where(x >= 0, x, NEG * x))


def _vloop(n, body, unroll=8):
    def f(j, carry):
        body(j)
        return carry
    lax.fori_loop(0, n, f, 0, unroll=unroll)


@functools.partial(
    pl.kernel,
    out_type=jax.ShapeDtypeStruct((E_PAD, AUX), _f32),
    mesh=_MESH,
    compiler_params=_NOTC,
    scratch_types=[
        pltpu.VMEM_SHARED((N_PAD, AUX), _f32),
        pltpu.VMEM((BLK, AUX), _f32),
        pltpu.VMEM((BLK, AUX), _f32),
        pltpu.VMEM((BLK, AUX), _f32),
        pltpu.VMEM((BLK, AUX), _f32),
        pltpu.VMEM((NB1 * SUB, CH), jnp.int32),
        pltpu.VMEM((NB2 * SUB, CH), jnp.int32),
        pltpu.SemaphoreType.DMA,
        pltpu.SemaphoreType.DMA,
        pltpu.SemaphoreType.DMA,
        pltpu.SemaphoreType.DMA,
    ],
)
def _stage_a(p_hbm, dst2d_hbm, zero_hbm, a_hbm,
             den_sh, p0, p1, d0, d1, dstA, dstB,
             semp0, semp1, semg0, semg1):
    c = lax.axis_index("c")
    s = lax.axis_index("s")
    wid = s * NC + c
    row0 = s * ROWS_PT
    pltpu.sync_copy(zero_hbm.at[pl.ds(row0, ROWS_PT)],
                    den_sh.at[pl.ds(row0, ROWS_PT)])
    pltpu.sync_copy(dst2d_hbm.at[pl.ds(s * NB1 * SUB, NB1 * SUB)], dstA)
    pltpu.sync_copy(dst2d_hbm.at[pl.ds(wid * NB2 * SUB, NB2 * SUB)], dstB)
    plsc.subcore_barrier()

    pbufs = (p0, p1)
    dbufs = (d0, d1)
    psems = (semp0, semp1)
    gsems = (semg0, semg1)

    def p_src1(i):
        return p_hbm.at[pl.ds((s * NB1 + i) * BLK, BLK)]

    pltpu.async_copy(p_src1(0), p0, semp0)
    pltpu.async_copy(p_src1(1), p1, semp1)

    def blk1(k, _):
        for b in (0, 1):
            i = 2 * k + b
            pb = pbufs[b]
            pltpu.make_async_copy(p_src1(i), pb, psems[b]).wait()

            def row(j):
                pb[j, :] = _leaky_exp(pb[j, :])

            _vloop(BLK, row)
            for u in range(SUB):
                pltpu.sync_copy(pb.at[pl.ds(u * CH, CH)],
                                den_sh.at[dstA.at[i * SUB + u]], add=True)

            @pl.when(i + 2 < NB1)
            def _():
                pltpu.async_copy(p_src1(i + 2), pb, psems[b])
        return 0

    lax.fori_loop(0, NB1 // 2, blk1, 0, unroll=False)
    plsc.subcore_barrier()

    def p_src2(i):
        return p_hbm.at[pl.ds((wid * NB2 + i) * BLK, BLK)]

    def gathers(i, b):
        for u in range(SUB):
            pltpu.async_copy(den_sh.at[dstB.at[i * SUB + u]],
                             dbufs[b].at[pl.ds(u * CH, CH)], gsems[b])

    pltpu.async_copy(p_src2(0), p0, semp0)
    gathers(0, 0)
    pltpu.async_copy(p_src2(1), p1, semp1)
    gathers(1, 1)

    def blk2(k, _):
        for b in (0, 1):
            i = 2 * k + b
            pb = pbufs[b]
            db = dbufs[b]
            pltpu.make_async_copy(p_src2(i), pb, psems[b]).wait()
            for u in range(SUB):
                pltpu.make_async_copy(den_sh.at[dstB.at[i * SUB + u]],
                                      db.at[pl.ds(u * CH, CH)],
                                      gsems[b]).wait()

            def row(j):
                pb[j, :] = _leaky_exp(pb[j, :]) / db[j, :]

            _vloop(BLK, row)
            pltpu.sync_copy(pb, a_hbm.at[pl.ds((wid * NB2 + i) * BLK, BLK)])

            @pl.when(i + 2 < NB2)
            def _():
                pltpu.async_copy(p_src2(i + 2), pb, psems[b])
                gathers(i + 2, b)
        return 0

    lax.fori_loop(0, NB2 // 2, blk2, 0, unroll=False)


def _mm_body(a_ref, wt_ref, b_ref, lo_ref, hi_ref):
    r = jnp.dot(a_ref[...], wt_ref[...], preferred_element_type=_f32)
    r = r + b_ref[...]
    lo_ref[...] = r[:, :HALF]
    hi_ref[...] = r[:, HALF:]


def _linear(a, wt, b):
    MB = 1024
    return pl.pallas_call(
        _mm_body,
        grid=(E_PAD // MB,),
        in_specs=[
            pl.BlockSpec((MB, AUX), lambda i: (i, 0)),
            pl.BlockSpec((AUX, OUT), lambda i: (0, 0)),
            pl.BlockSpec((1, OUT), lambda i: (0, 0)),
        ],
        out_specs=[
            pl.BlockSpec((MB, HALF), lambda i: (i, 0)),
            pl.BlockSpec((MB, HALF), lambda i: (i, 0)),
        ],
        out_shape=[
            jax.ShapeDtypeStruct((E_PAD, HALF), _f32),
            jax.ShapeDtypeStruct((E_PAD, HALF), _f32),
        ],
    )(a, wt, b)


@functools.partial(
    pl.kernel,
    out_type=(
        jax.ShapeDtypeStruct((N_PAD, HALF), _f32),
        jax.ShapeDtypeStruct((N_PAD, HALF), _f32),
    ),
    mesh=_MESH,
    compiler_params=_NOTC,
    scratch_types=[
        pltpu.VMEM_SHARED((N_PAD, HALF), _f32),
        pltpu.VMEM((CC, HALF), _f32),
        pltpu.VMEM((CC, HALF), _f32),
        pltpu.VMEM((CC, HALF), _f32),
        pltpu.VMEM((CC, HALF), _f32),
        pltpu.VMEM((NCC, CC), jnp.int32),
        pltpu.VMEM((CC,), jnp.int32),
        pltpu.VMEM((CC,), jnp.int32),
        pltpu.VMEM((CC,), jnp.int32),
        pltpu.VMEM((CC,), jnp.int32),
        pltpu.SemaphoreType.DMA,
        pltpu.SemaphoreType.DMA,
        pltpu.SemaphoreType.DMA,
        pltpu.SemaphoreType.DMA,
    ],
)
def _stage_c(srclo_hbm, srchi_hbm, alo_hbm, ahi_hbm, packed_hbm,
             zero_hbm, outlo_hbm, outhi_hbm,
             acc_sh, r0, r1, a0, a1, packed_all, s0, s1, t0, t1,
             semr0, semr1, sema0, sema1):
    c = lax.axis_index("c")
    s = lax.axis_index("s")
    row0 = s * ROWS_PT
    pltpu.sync_copy(zero_hbm.at[pl.ds(row0, ROWS_PT)],
                    acc_sh.at[pl.ds(row0, ROWS_PT)])
    pltpu.sync_copy(packed_hbm.at[pl.ds(s * NCC, NCC)], packed_all)
    plsc.subcore_barrier()

    rbufs = (r0, r1)
    abufs = (a0, a1)
    sbufs = (s0, s1)
    tbufs = (t0, t1)
    rsems = (semr0, semr1)
    asems = (sema0, sema1)

    def unpack(i, b):
        for q in range(CC // L):
            sl = pl.ds(q * L, L)
            pk = packed_all[i, sl]
            sbufs[b][sl] = lax.shift_right_logical(pk, 14)
            tbufs[b][sl] = lax.bitwise_and(pk, PACK - 1)

    def run(tbl_hbm, a_half_hbm):
        def g_src(i, b):
            return tbl_hbm.at[sbufs[b]]

        def a_src(i):
            return a_half_hbm.at[pl.ds((s * NCC + i) * CC, CC)]

        unpack(0, 0)
        pltpu.async_copy(g_src(0, 0), r0, semr0)
        pltpu.async_copy(a_src(0), a0, sema0)
        unpack(1, 1)
        pltpu.async_copy(g_src(1, 1), r1, semr1)
        pltpu.async_copy(a_src(1), a1, sema1)

        def chunk(k, _):
            for b in (0, 1):
                i = 2 * k + b
                rb = rbufs[b]
                ab = abufs[b]
                pltpu.make_async_copy(g_src(i, b), rb, rsems[b]).wait()
                pltpu.make_async_copy(a_src(i), ab, asems[b]).wait()

                def row(j):
                    for q in range(HALF // L):
                        sl = pl.ds(q * L, L)
                        rb[j, sl] = rb[j, sl] * ab[j, sl]

                _vloop(CC, row, unroll=2)
                pltpu.sync_copy(rb, acc_sh.at[tbufs[b]], add=True)

                @pl.when(i + 2 < NCC)
                def _():
                    unpack(i + 2, b)
                    pltpu.async_copy(g_src(i + 2, b), rb, rsems[b])
                    pltpu.async_copy(a_src(i + 2), ab, asems[b])
            return 0

        lax.fori_loop(0, NCC // 2, chunk, 0, unroll=False)

    @pl.when(c == 0)
    def _():
        run(srclo_hbm, alo_hbm)

    @pl.when(c == 1)
    def _():
        run(srchi_hbm, ahi_hbm)

    plsc.subcore_barrier()

    @pl.when(c == 0)
    def _():
        pltpu.sync_copy(acc_sh.at[pl.ds(row0, ROWS_PT)],
                        outlo_hbm.at[pl.ds(row0, ROWS_PT)])

    @pl.when(c == 1)
    def _():
        pltpu.sync_copy(acc_sh.at[pl.ds(row0, ROWS_PT)],
                        outhi_hbm.at[pl.ds(row0, ROWS_PT)])


def kernel(src_ft, dst_ft, edge_p, edge_index, W_pos_w, W_pos_b):
    src = edge_index[0]
    dst = edge_index[1]
    pad = E_PAD - E
    p_pad = jnp.pad(edge_p, ((0, pad), (0, 0)))
    src_pad = jnp.pad(src, (0, pad))
    dst_pad = jnp.pad(dst, (0, pad), constant_values=N)
    dst2d = dst_pad.reshape(E_PAD // CH, CH)
    packed2d = (src_pad * PACK + dst_pad).reshape(E_PAD // CC, CC)
    zero_aux = jnp.zeros((N_PAD, AUX), _f32)
    zero_half = jnp.zeros((N_PAD, HALF), _f32)

    a = _stage_a(p_pad, dst2d, zero_aux)
    a_lo, a_hi = _linear(a, W_pos_w.T, W_pos_b[None, :])
    out_lo, out_hi = _stage_c(src_ft[:, :HALF], src_ft[:, HALF:],
                              a_lo, a_hi, packed2d, zero_half)
    out = jnp.concatenate([out_lo[:N], out_hi[:N]], axis=1)
    return out[:, None, :]
